# baseline (speedup 1.0000x reference)
import functools

import jax
import jax.numpy as jnp
from jax import lax
from jax.experimental import pallas as pl
from jax.experimental.pallas import tpu as pltpu
from jax.experimental.pallas import tpu_sc as plsc

B, V, K = 128, 100000, 64
VPAD = 100096
NT = 49
CW = NT * 128
HALF = 8 * CW
_CHUNKS0 = tuple((j * CW, NT) for j in range(8))
_CHUNKS1 = tuple((HALF + j * CW, NT) for j in range(7)) + ((HALF + 7 * CW, 47),)


def _sc_mask_kernel(scores_hbm, allowed_hbm, out_hbm,
                    bufa, bufb, alw, slab, vals, gsem, gsem2, fs0, fs1):
    c = lax.axis_index("c")
    s = lax.axis_index("s")
    wid = c * 16 + s
    g = wid % 16
    half = wid // 16
    row0 = pl.multiple_of(g * 8, 8)

    pltpu.sync_copy(allowed_hbm.at[pl.ds(row0, 8)], alw)

    neg = jnp.full((16,), -jnp.inf, dtype=jnp.float32)
    lane = lax.iota(jnp.int32, 16)

    def fill(buf):
        def ftile(t, carry):
            for r in range(8):
                for i in range(128 // 16):
                    buf[t, r, pl.ds(i * 16, 16)] = neg
            return carry
        lax.fori_loop(0, NT, ftile, 0)

    def fire_slabs(r, sl):
        sem = gsem if sl == 0 else gsem2
        handles = []
        for q in range(K // 16):
            id16 = alw[r, pl.ds(q * 16, 16)]
            for j in range(16):
                idv = id16[j]
                off = pl.multiple_of((idv >> 7) * 128, 128)
                src = scores_hbm.at[row0 + r].at[pl.ds(off, 128)]
                handles.append(
                    pltpu.async_copy(src, slab.at[sl * K + q * 16 + j], sem))
        return handles

    def extract(r, sl):
        for q in range(K // 16):
            id16 = alw[r, pl.ds(q * 16, 16)]
            off16 = jnp.bitwise_and(id16, 127)
            vals[r, pl.ds(q * 16, 16)] = plsc.load_gather(
                slab, [lane + (sl * K + q * 16), off16])

    h_prev = fire_slabs(0, 0)
    fill(bufa)
    for r in range(1, 8):
        h_cur = fire_slabs(r, r % 2)
        for h in h_prev:
            h.wait()
        extract(r - 1, (r - 1) % 2)
        h_prev = h_cur
    for h in h_prev:
        h.wait()
    extract(7, 1)

    def patch(buf, c0, ntiles, restore):
        tbase = c0 // 128

        def body(r, carry):
            r16 = jnp.broadcast_to(r, (16,)).astype(jnp.int32)
            for q in range(K // 16):
                id16 = alw[r, pl.ds(q * 16, 16)]
                t16 = (id16 >> 7) - tbase
                l16 = jnp.bitwise_and(id16, 127)
                m = (t16 >= 0) & (t16 < ntiles)
                v16 = neg if restore else vals[r, pl.ds(q * 16, 16)]
                plsc.store_scatter(buf, [t16, r16, l16], v16, mask=m)
            return carry

        lax.fori_loop(0, 8, body, 0)

    for hsel, chunk_list in ((0, _CHUNKS0), (1, _CHUNKS1)):
        @pl.when(half == hsel)
        def _(chunk_list=chunk_list):
            bufs = (bufa, bufb)
            sems = (fs0, fs1)
            pending = [None, None]
            pend_chunk = [None, None]
            for ci, (c0, ntiles) in enumerate(chunk_list):
                slot = ci % 2
                buf = bufs[slot]
                if pending[slot] is not None:
                    pending[slot].wait()
                    pc0, pnt = pend_chunk[slot]
                    patch(buf, pc0, pnt, restore=True)
                patch(buf, c0, ntiles, restore=False)

                def fire(t, carry, buf=buf, c0=c0, sem=sems[slot]):
                    col = pl.multiple_of(c0 + t * 128, 128)
                    dst = out_hbm.at[pl.ds(row0, 8), pl.ds(col, 128)]
                    pltpu.async_copy(buf.at[t], dst, sem)
                    return carry

                lax.fori_loop(0, ntiles, fire, 0)
                if ci == 0:
                    fill(bufb)
                c0d = pl.multiple_of(c0 + 0 * wid, 128)
                span = out_hbm.at[pl.ds(row0, 8), pl.ds(c0d, ntiles * 128)]
                src_dummy = scores_hbm.at[pl.ds(row0, 8), pl.ds(c0d, ntiles * 128)]
                pending[slot] = pltpu.make_async_copy(src_dummy, span, sems[slot])
                pend_chunk[slot] = (c0, ntiles)
            for slot in (0, 1):
                if pending[slot] is not None:
                    pending[slot].wait()


@jax.jit
def _masked_scores(scores, allowed_ids):
    mesh = plsc.VectorSubcoreMesh(core_axis_name="c", subcore_axis_name="s")
    run = functools.partial(
        pl.kernel,
        out_type=jax.ShapeDtypeStruct((B, V), jnp.float32),
        mesh=mesh,
        compiler_params=pltpu.CompilerParams(needs_layout_passes=False),
        scratch_types=[
            pltpu.VMEM((NT, 8, 128), jnp.float32),
            pltpu.VMEM((NT, 8, 128), jnp.float32),
            pltpu.VMEM((8, K), jnp.int32),
            pltpu.VMEM((2 * K, 128), jnp.float32),
            pltpu.VMEM((8, K), jnp.float32),
            pltpu.SemaphoreType.DMA,
            pltpu.SemaphoreType.DMA,
            pltpu.SemaphoreType.DMA,
            pltpu.SemaphoreType.DMA,
        ],
    )(_sc_mask_kernel)
    return run(scores, allowed_ids)


def kernel(input_ids, scores, allowed_ids):
    del input_ids
    return _masked_scores(scores, allowed_ids)
